# baseline (speedup 1.0000x reference)
import functools

import jax
import jax.numpy as jnp
from jax import lax
from jax.experimental import pallas as pl
from jax.experimental.pallas import tpu as pltpu
from jax.experimental.pallas import tpu_sc as plsc

_B, _H, _S, _D, _Q = 8, 16, 4096, 128, 8
_BH = _B * _H
_NW = 32
_BH_PW = _BH // _NW
_SP = _S // 2
_LOW = 65535
_HIGH = -65536

_mesh = plsc.VectorSubcoreMesh(core_axis_name="c", subcore_axis_name="s")


def _merge_rows(buf_p, buf_u, posbit16, qbit):
    def body(j, _):
        for c in range(_D // 16):
            o = buf_p[j, pl.ds(c * 16, 16)]
            u = buf_u[j, pl.ds(c * 16, 16)]
            if qbit == 0:
                val = jnp.bitwise_and(u, _LOW)
            else:
                val = lax.shift_right_logical(u, 16)
            lo_merged = jnp.bitwise_or(jnp.bitwise_and(o, _HIGH), val)
            hi_merged = jnp.bitwise_or(jnp.bitwise_and(o, _LOW),
                                       lax.shift_left(val, 16))
            buf_p[j, pl.ds(c * 16, 16)] = jnp.where(
                posbit16 == 0, lo_merged, hi_merged)
        return 0
    lax.fori_loop(0, 16, body, 0)


@functools.partial(
    pl.kernel,
    out_type=(),
    mesh=_mesh,
    scratch_types=[
        pltpu.VMEM((_Q, 16), jnp.int32),
        pltpu.VMEM((16,), jnp.int32),
        pltpu.VMEM((16,), jnp.int32),
        pltpu.VMEM((16, _D), jnp.int32),
        pltpu.VMEM((16, _D), jnp.int32),
        pltpu.VMEM((16, _D), jnp.int32),
        pltpu.VMEM((16, _D), jnp.int32),
        pltpu.SemaphoreType.DMA,
    ],
)
def _sc_scatter(pos_hbm, kval_hbm, vval_hbm, kout_ref, vout_ref,
                pos_b, idx_p, idx_u, buf_p_k, buf_p_v, buf_u_k, buf_u_v, sem):
    cid = lax.axis_index("c")
    sid = lax.axis_index("s")
    wid = sid * 2 + cid
    pltpu.async_copy(pos_hbm, pos_b, sem).wait()
    lanes = lax.broadcasted_iota(jnp.int32, (16,), 0)
    bh_l = wid * _BH_PW + jnp.bitwise_and(lanes, 3)
    ko32 = kout_ref.bitcast(jnp.int32)
    vo32 = vout_ref.bitcast(jnp.int32)
    ku32 = kval_hbm.bitcast(jnp.int32)
    vu32 = vval_hbm.bitcast(jnp.int32)
    for q in range(_Q):
        pos16 = pos_b[q, :]
        posbit16 = jnp.bitwise_and(pos16, 1)
        idx_p[...] = bh_l * _SP + lax.shift_right_logical(pos16, 1)
        idx_u[...] = bh_l * (_Q // 2) + (q // 2)
        cpk = pltpu.async_copy(ko32.at[idx_p], buf_p_k, sem)
        cpv = pltpu.async_copy(vo32.at[idx_p], buf_p_v, sem)
        cuk = pltpu.async_copy(ku32.at[idx_u], buf_u_k, sem)
        cuv = pltpu.async_copy(vu32.at[idx_u], buf_u_v, sem)
        cpk.wait()
        cpv.wait()
        cuk.wait()
        cuv.wait()
        _merge_rows(buf_p_k, buf_u_k, posbit16, q % 2)
        _merge_rows(buf_p_v, buf_u_v, posbit16, q % 2)
        cpk = pltpu.async_copy(buf_p_k, ko32.at[idx_p], sem)
        cpv = pltpu.async_copy(buf_p_v, vo32.at[idx_p], sem)
        cpk.wait()
        cpv.wait()


def kernel(k_cache, v_cache, input_pos, k_val, v_val):
    pos_b = jnp.broadcast_to(input_pos[:, None], (_Q, 16))
    kv2 = k_val.reshape(_BH * _Q, _D)
    vv2 = v_val.reshape(_BH * _Q, _D)
    kr = jax.new_ref(k_cache.reshape(_BH * _S, _D))
    vr = jax.new_ref(v_cache.reshape(_BH * _S, _D))
    _sc_scatter(pos_b, kv2, vv2, kr, vr)
    K = kr[...].reshape(_B, _H, _S, _D)
    V = vr[...].reshape(_B, _H, _S, _D)
    return (K, K, V)


# speedup vs baseline: 1.1454x; 1.1454x over previous
import jax
import jax.numpy as jnp
from jax.experimental import pallas as pl
from jax.experimental.pallas import tpu as pltpu

_B, _H, _S, _D, _Q = 8, 16, 4096, 128, 8
_BH = _B * _H
_BHB = 4
_W = 8


def _fused_kernel(pos_ref, kc_ref, vc_ref, kv_ref, vv_ref, ko_ref, vo_ref):
    ko_ref[...] = kc_ref[...]
    vo_ref[...] = vc_ref[...]
    sub_iota = jax.lax.broadcasted_iota(jnp.int32, (1, _W, 1), 1)
    for q in range(_Q):
        pos = pos_ref[q]
        wb = pl.multiple_of((pos // _W) * _W, _W)
        r = pos % _W
        mask = sub_iota == r
        ko_ref[:, pl.ds(wb, _W), :] = jnp.where(
            mask, kv_ref[:, q:q + 1, :], ko_ref[:, pl.ds(wb, _W), :])
        vo_ref[:, pl.ds(wb, _W), :] = jnp.where(
            mask, vv_ref[:, q:q + 1, :], vo_ref[:, pl.ds(wb, _W), :])


def kernel(k_cache, v_cache, input_pos, k_val, v_val):
    kc = k_cache.reshape(_BH, _S, _D)
    vc = v_cache.reshape(_BH, _S, _D)
    kv = k_val.reshape(_BH, _Q, _D)
    vv = v_val.reshape(_BH, _Q, _D)
    grid = (_BH // _BHB,)
    cache_spec = pl.BlockSpec((_BHB, _S, _D), lambda i: (i, 0, 0))
    val_spec = pl.BlockSpec((_BHB, _Q, _D), lambda i: (i, 0, 0))
    ko, vo = pl.pallas_call(
        _fused_kernel,
        grid=grid,
        in_specs=[
            pl.BlockSpec(memory_space=pltpu.SMEM),
            cache_spec, cache_spec, val_spec, val_spec,
        ],
        out_specs=(cache_spec, cache_spec),
        out_shape=(jax.ShapeDtypeStruct((_BH, _S, _D), kc.dtype),
                   jax.ShapeDtypeStruct((_BH, _S, _D), vc.dtype)),
    )(input_pos, kc, vc, kv, vv)
    K = ko.reshape(_B, _H, _S, _D)
    V = vo.reshape(_B, _H, _S, _D)
    return (K, K, V)
